# Initial kernel scaffold; baseline (speedup 1.0000x reference)
import jax
import jax.numpy as jnp
from jax.experimental import pallas as pl


def kernel(x, edge_index, beta):
    raise NotImplementedError("write your pallas kernel here")


# speedup vs baseline: 10.5427x; 10.5427x over previous
import functools

import jax
import jax.numpy as jnp
from jax import lax
from jax.experimental import pallas as pl
from jax.experimental.pallas import tpu as pltpu
from jax.experimental.pallas import tpu_sc as plsc

D = 128
TAIL = 16
DP = D + TAIL
LANES = 16
BN = 1000


def _norm_body(x_ref, out_ref):
    x = x_ref[...]
    nrm = jnp.sqrt(jnp.sum(x * x, axis=1, keepdims=True))
    xn = x / (nrm + 1e-12)
    tail = jnp.broadcast_to(nrm, (x.shape[0], TAIL))
    out_ref[...] = jnp.concatenate([xn, tail], axis=1)


def _combine_body(a0_ref, a1_ref, out_ref):
    s = a0_ref[...] + a1_ref[...]
    out_ref[...] = s[:, :D] / (s[:, D:D + 1] + 1e-16)


@functools.lru_cache(maxsize=None)
def _make_sc(N, E):
    info = plsc.get_sparse_core_info()
    NC, NS = info.num_cores, info.num_subcores
    NW = NC * NS
    EPW = E // NW
    B = 80
    NCHUNK = EPW // B
    RPT = N // NS
    NZ = RPT // B
    REM = RPT - NZ * B
    mesh = plsc.VectorSubcoreMesh(core_axis_name="c", subcore_axis_name="s")

    @functools.partial(
        pl.kernel,
        out_type=jax.ShapeDtypeStruct((NC * N, DP), jnp.float32),
        mesh=mesh,
        compiler_params=pltpu.CompilerParams(use_tc_tiling_on_sc=False,
                                             needs_layout_passes=False),
        scratch_types=[
            pltpu.VMEM((B,), jnp.int32),
            pltpu.VMEM((B,), jnp.int32),
            pltpu.VMEM((B, DP), jnp.float32),
            pltpu.VMEM((B, DP), jnp.float32),
            pltpu.VMEM((B, DP), jnp.float32),
            pltpu.VMEM((LANES,), jnp.float32),
            pltpu.VMEM_SHARED((N, DP), jnp.float32),
            pltpu.SemaphoreType.DMA,
            pltpu.SemaphoreType.DMA,
        ],
    )
    def sc(table, srcs, dsts, beta16, out,
           sidx, didx, srows, drows, obuf, bvec, acc, sem1, sem2):
        c = lax.axis_index("c")
        s = lax.axis_index("s")
        wid = c * NS + s
        zero16 = jnp.zeros((LANES,), jnp.float32)

        @pl.loop(0, B)
        def _zero_obuf(r):
            for k in range(DP // LANES):
                obuf[r, pl.ds(k * LANES, LANES)] = zero16

        row0 = s * RPT
        for j in range(NZ):
            pltpu.sync_copy(obuf, acc.at[pl.ds(row0 + j * B, B)])
        if REM:
            pltpu.sync_copy(obuf.at[pl.ds(0, REM)],
                            acc.at[pl.ds(row0 + NZ * B, REM)])
        pltpu.sync_copy(beta16, bvec)
        plsc.subcore_barrier()

        bs = jnp.max(bvec[...])
        lane = lax.iota(jnp.int32, LANES)
        oh0 = (lane == 0).astype(jnp.float32)
        ebase = wid * EPW

        @pl.loop(0, NCHUNK)
        def _chunk(ci):
            base = ebase + ci * B
            pltpu.sync_copy(srcs.at[pl.ds(base, B)], sidx)
            pltpu.sync_copy(dsts.at[pl.ds(base, B)], didx)
            pltpu.async_copy(table.at[sidx], srows, sem1).wait()
            pltpu.async_copy(table.at[didx], drows, sem2).wait()

            @pl.loop(0, B)
            def _edge(e):
                a = [srows[e, pl.ds(k * LANES, LANES)]
                     for k in range(D // LANES)]
                b = [drows[e, pl.ds(k * LANES, LANES)]
                     for k in range(D // LANES)]
                accv = a[0] * b[0]
                for k in range(1, D // LANES):
                    accv = accv + a[k] * b[k]
                dot = jnp.sum(accv)
                wv = jnp.exp(jnp.full((LANES,), bs * dot))
                normv = srows[e, pl.ds(D, LANES)]
                sv = wv * normv
                for k in range(D // LANES):
                    obuf[e, pl.ds(k * LANES, LANES)] = a[k] * sv
                obuf[e, pl.ds(D, LANES)] = wv * oh0

            pltpu.sync_copy(obuf, acc.at[didx], add=True)

        plsc.subcore_barrier()
        pltpu.sync_copy(acc.at[pl.ds(row0, RPT)],
                        out.at[pl.ds(c * N + row0, RPT)])

    return sc


def kernel(x, edge_index, beta):
    N = x.shape[0]
    E = edge_index.shape[1]
    table = pl.pallas_call(
        _norm_body,
        grid=(N // BN,),
        in_specs=[pl.BlockSpec((BN, D), lambda i: (i, 0))],
        out_specs=pl.BlockSpec((BN, DP), lambda i: (i, 0)),
        out_shape=jax.ShapeDtypeStruct((N, DP), jnp.float32),
    )(x)
    src = edge_index[0]
    dst = edge_index[1]
    beta16 = jnp.broadcast_to(beta.astype(jnp.float32), (LANES,))
    accflat = _make_sc(N, E)(table, src, dst, beta16)
    nb = N // BN
    out = pl.pallas_call(
        _combine_body,
        grid=(nb,),
        in_specs=[pl.BlockSpec((BN, DP), lambda i: (i, 0)),
                  pl.BlockSpec((BN, DP), lambda i: (i + nb, 0))],
        out_specs=pl.BlockSpec((BN, D), lambda i: (i, 0)),
        out_shape=jax.ShapeDtypeStruct((N, D), jnp.float32),
    )(accflat, accflat)
    return out


# speedup vs baseline: 11.9497x; 1.1335x over previous
import functools

import jax
import jax.numpy as jnp
from jax import lax
from jax.experimental import pallas as pl
from jax.experimental.pallas import tpu as pltpu
from jax.experimental.pallas import tpu_sc as plsc

D = 128
TAIL = 16
DP = D + TAIL
LANES = 16
BN = 1000
B = 40
PAIRB = 2 * B
BSC = 48


def _norm_body(x_ref, out_ref):
    x = x_ref[...]
    nrm = jnp.sqrt(jnp.sum(x * x, axis=1, keepdims=True))
    xn = x / (nrm + 1e-12)
    tail = jnp.broadcast_to(nrm, (x.shape[0], TAIL))
    out_ref[...] = jnp.concatenate([xn, tail], axis=1)


def _combine_body(a0_ref, a1_ref, out_ref):
    s = a0_ref[...] + a1_ref[...]
    out_ref[...] = s[:, :D] / (s[:, D:D + 1] + 1e-16)


@functools.lru_cache(maxsize=None)
def _make_sc(N, E):
    info = plsc.get_sparse_core_info()
    NC, NS = info.num_cores, info.num_subcores
    NW = NC * NS
    EPW = E // NW
    NCHUNK = EPW // B
    NPAIR = NCHUNK // 2
    NQUAD = (NCHUNK - 2) // 4
    RPT = N // NS
    NZ = RPT // BSC
    REM = RPT - NZ * BSC
    mesh = plsc.VectorSubcoreMesh(core_axis_name="c", subcore_axis_name="s")

    @functools.partial(
        pl.kernel,
        out_type=jax.ShapeDtypeStruct((NC * N, DP), jnp.float32),
        mesh=mesh,
        compiler_params=pltpu.CompilerParams(use_tc_tiling_on_sc=False,
                                             needs_layout_passes=False),
        scratch_types=[
            pltpu.VMEM((PAIRB,), jnp.int32),
            pltpu.VMEM((PAIRB,), jnp.int32),
            pltpu.VMEM((PAIRB,), jnp.int32),
            pltpu.VMEM((PAIRB,), jnp.int32),
            pltpu.VMEM((B, DP), jnp.float32),
            pltpu.VMEM((B, DP), jnp.float32),
            pltpu.VMEM((B, DP), jnp.float32),
            pltpu.VMEM((B, DP), jnp.float32),
            pltpu.VMEM((BSC, DP), jnp.float32),
            pltpu.VMEM((BSC, DP), jnp.float32),
            pltpu.VMEM((BSC,), jnp.int32),
            pltpu.VMEM((BSC,), jnp.int32),
            pltpu.VMEM((LANES,), jnp.float32),
            pltpu.VMEM_SHARED((N, DP), jnp.float32),
            pltpu.SemaphoreType.DMA,
            pltpu.SemaphoreType.DMA,
            pltpu.SemaphoreType.DMA,
            pltpu.SemaphoreType.DMA,
            pltpu.SemaphoreType.DMA,
            pltpu.SemaphoreType.DMA,
            pltpu.SemaphoreType.DMA,
            pltpu.SemaphoreType.DMA,
        ],
    )
    def sc(table, srcs3, dsts3, beta16, out,
           ips0, ipd0, ips1, ipd1, srows0, drows0, srows1, drows1,
           obuf0, obuf1, sdidx0, sdidx1, bvec, acc,
           gs0, gd0, gs1, gd1, o0, o1, i0, i1):
        c = lax.axis_index("c")
        s = lax.axis_index("s")
        wid = c * NS + s
        zero16 = jnp.zeros((LANES,), jnp.float32)
        rows = ((srows0, drows0, gs0, gd0), (srows1, drows1, gs1, gd1))
        obufs = ((obuf0, o0, sdidx0), (obuf1, o1, sdidx1))
        ipairs = ((ips0, ipd0, i0), (ips1, ipd1, i1))

        @pl.loop(0, BSC)
        def _zero_obuf(r):
            for k in range(DP // LANES):
                obuf0[r, pl.ds(k * LANES, LANES)] = zero16
                obuf1[r, pl.ds(k * LANES, LANES)] = zero16

        lane = lax.iota(jnp.int32, LANES)

        def copy_sdidx(ipd, half, sd):
            for t in range(2):
                sd[pl.ds(16 * t, 16)] = ipd[pl.ds(half * B + 16 * t, 16)]
            gidx = jnp.minimum(lane + (half * B + 32), half * B + B - 1)
            sd[pl.ds(32, 16)] = plsc.load_gather(ipd, [gidx])

        row0 = s * RPT
        for j in range(NZ):
            pltpu.sync_copy(obuf0, acc.at[pl.ds(row0 + j * BSC, BSC)])
        if REM:
            pltpu.sync_copy(obuf0.at[pl.ds(0, REM)],
                            acc.at[pl.ds(row0 + NZ * BSC, REM)])
        pltpu.sync_copy(beta16, bvec)

        pltpu.sync_copy(srcs3.at[wid, 0], ips0)
        pltpu.sync_copy(dsts3.at[wid, 0], ipd0)
        pltpu.async_copy(srcs3.at[wid, 1], ips1, i1)
        pltpu.async_copy(dsts3.at[wid, 1], ipd1, i1)
        copy_sdidx(ipd0, 0, sdidx0)
        copy_sdidx(ipd0, 0, sdidx1)
        pltpu.async_copy(obuf0, acc.at[sdidx0], o0, add=True)
        pltpu.async_copy(obuf1, acc.at[sdidx1], o1, add=True)
        pltpu.async_copy(table.at[ips0.at[pl.ds(0, B)]], srows0, gs0)
        pltpu.async_copy(table.at[ipd0.at[pl.ds(0, B)]], drows0, gd0)
        plsc.subcore_barrier()

        bs = jnp.max(bvec[...])
        oh0 = (lane == 0).astype(jnp.float32)

        def g_issue(ip, half, p):
            ips, ipd, _ = ip
            sr, dr, ss, sd = rows[p]
            pltpu.async_copy(table.at[ips.at[pl.ds(half * B, B)]], sr, ss)
            pltpu.async_copy(table.at[ipd.at[pl.ds(half * B, B)]], dr, sd)

        def g_wait(ip, half, p):
            ips, ipd, _ = ip
            sr, dr, ss, sd = rows[p]
            pltpu.make_async_copy(table.at[ips.at[pl.ds(half * B, B)]],
                                  sr, ss).wait()
            pltpu.make_async_copy(table.at[ipd.at[pl.ds(half * B, B)]],
                                  dr, sd).wait()

        def i_issue(ip, pairno):
            ips, ipd, isem = ip
            pltpu.async_copy(srcs3.at[wid, pairno], ips, isem)
            pltpu.async_copy(dsts3.at[wid, pairno], ipd, isem)

        def i_wait(ip, pairno):
            ips, ipd, isem = ip
            pltpu.make_async_copy(srcs3.at[wid, pairno], ips, isem).wait()
            pltpu.make_async_copy(dsts3.at[wid, pairno], ipd, isem).wait()

        def compute(ip, half, p):
            _, ipd, _ = ip
            sr, dr, _, _ = rows[p]
            ob, osem, sd = obufs[p]
            pltpu.make_async_copy(ob, acc.at[sd], osem).wait()
            copy_sdidx(ipd, half, sd)

            @pl.loop(0, B, unroll=2)
            def _edge(e):
                a = [sr[e, pl.ds(k * LANES, LANES)]
                     for k in range(D // LANES)]
                b = [dr[e, pl.ds(k * LANES, LANES)]
                     for k in range(D // LANES)]
                accv = a[0] * b[0]
                for k in range(1, D // LANES):
                    accv = accv + a[k] * b[k]
                dot = jnp.sum(accv)
                wv = jnp.exp(jnp.full((LANES,), bs * dot))
                normv = sr[e, pl.ds(D, LANES)]
                sv = wv * normv
                for k in range(D // LANES):
                    ob[e, pl.ds(k * LANES, LANES)] = a[k] * sv
                ob[e, pl.ds(D, LANES)] = wv * oh0

            pltpu.async_copy(ob, acc.at[sd], osem, add=True)

        @pl.loop(0, NQUAD)
        def _quad(q):
            ip0, ip1 = ipairs
            g_issue(ip0, 1, 1)
            g_wait(ip0, 0, 0)
            compute(ip0, 0, 0)
            i_wait(ip1, 2 * q + 1)
            g_issue(ip1, 0, 0)
            g_wait(ip0, 1, 1)
            compute(ip0, 1, 1)
            i_issue(ip0, 2 * q + 2)
            g_issue(ip1, 1, 1)
            g_wait(ip1, 0, 0)
            compute(ip1, 0, 0)
            i_wait(ip0, 2 * q + 2)
            g_issue(ip0, 0, 0)
            g_wait(ip1, 1, 1)
            compute(ip1, 1, 1)

            @pl.when(2 * q + 3 < NPAIR)
            def _():
                i_issue(ip1, 2 * q + 3)

        ip0 = ipairs[0]
        g_issue(ip0, 1, 1)
        g_wait(ip0, 0, 0)
        compute(ip0, 0, 0)
        g_wait(ip0, 1, 1)
        compute(ip0, 1, 1)
        pltpu.make_async_copy(obuf0, acc.at[sdidx0], o0).wait()
        pltpu.make_async_copy(obuf1, acc.at[sdidx1], o1).wait()
        plsc.subcore_barrier()
        pltpu.sync_copy(acc.at[pl.ds(row0, RPT)],
                        out.at[pl.ds(c * N + row0, RPT)])

    return sc


def kernel(x, edge_index, beta):
    N = x.shape[0]
    E = edge_index.shape[1]
    NW = 32
    table = pl.pallas_call(
        _norm_body,
        grid=(N // BN,),
        in_specs=[pl.BlockSpec((BN, D), lambda i: (i, 0))],
        out_specs=pl.BlockSpec((BN, DP), lambda i: (i, 0)),
        out_shape=jax.ShapeDtypeStruct((N, DP), jnp.float32),
    )(x)
    npair = (E // NW) // PAIRB
    srcs3 = edge_index[0].reshape(NW, npair, PAIRB)
    dsts3 = edge_index[1].reshape(NW, npair, PAIRB)
    beta16 = jnp.broadcast_to(beta.astype(jnp.float32), (LANES,))
    accflat = _make_sc(N, E)(table, srcs3, dsts3, beta16)
    nb = N // BN
    out = pl.pallas_call(
        _combine_body,
        grid=(nb,),
        in_specs=[pl.BlockSpec((BN, DP), lambda i: (i, 0)),
                  pl.BlockSpec((BN, DP), lambda i: (i + nb, 0))],
        out_specs=pl.BlockSpec((BN, D), lambda i: (i, 0)),
        out_shape=jax.ShapeDtypeStruct((N, D), jnp.float32),
    )(accflat, accflat)
    return out


# speedup vs baseline: 12.1343x; 1.0155x over previous
import functools

import jax
import jax.numpy as jnp
from jax import lax
from jax.experimental import pallas as pl
from jax.experimental.pallas import tpu as pltpu
from jax.experimental.pallas import tpu_sc as plsc

D = 128
TAIL = 16
DP = D + TAIL
PW = D // 2
TW = PW + TAIL
LANES = 16
BN = 1000
B = 40
PAIRB = 2 * B
BSC = 48


def _norm_body(x_ref, out_ref):
    x = x_ref[...]
    nrm = jnp.sqrt(jnp.sum(x * x, axis=1, keepdims=True))
    xn = x / (nrm + 1e-12)
    u = jax.lax.bitcast_convert_type(xn.astype(jnp.bfloat16),
                                     jnp.uint16).astype(jnp.uint32)
    groups = []
    for k in range(4):
        lo = u[:, 32 * k:32 * k + 16]
        hi = u[:, 32 * k + 16:32 * k + 32]
        groups.append((hi << 16) | lo)
    packed = jax.lax.bitcast_convert_type(
        jnp.concatenate(groups, axis=1), jnp.float32)
    tail = jnp.broadcast_to(nrm, (x.shape[0], TAIL))
    out_ref[...] = jnp.concatenate([packed, tail], axis=1)


def _combine_body(a0_ref, a1_ref, out_ref):
    s = a0_ref[...] + a1_ref[...]
    out_ref[...] = s[:, :D] / (s[:, D:D + 1] + 1e-16)


@functools.lru_cache(maxsize=None)
def _make_sc(N, E):
    info = plsc.get_sparse_core_info()
    NC, NS = info.num_cores, info.num_subcores
    NW = NC * NS
    EPW = E // NW
    NCHUNK = EPW // B
    NPAIR = NCHUNK // 2
    NQUAD = (NCHUNK - 2) // 4
    RPT = N // NS
    NZ = RPT // BSC
    REM = RPT - NZ * BSC
    mesh = plsc.VectorSubcoreMesh(core_axis_name="c", subcore_axis_name="s")

    @functools.partial(
        pl.kernel,
        out_type=jax.ShapeDtypeStruct((NC * N, DP), jnp.float32),
        mesh=mesh,
        compiler_params=pltpu.CompilerParams(use_tc_tiling_on_sc=False,
                                             needs_layout_passes=False),
        scratch_types=[
            pltpu.VMEM((PAIRB,), jnp.int32),
            pltpu.VMEM((PAIRB,), jnp.int32),
            pltpu.VMEM((PAIRB,), jnp.int32),
            pltpu.VMEM((PAIRB,), jnp.int32),
            pltpu.VMEM((B, TW), jnp.float32),
            pltpu.VMEM((B, TW), jnp.float32),
            pltpu.VMEM((B, TW), jnp.float32),
            pltpu.VMEM((B, TW), jnp.float32),
            pltpu.VMEM((BSC, DP), jnp.float32),
            pltpu.VMEM((BSC, DP), jnp.float32),
            pltpu.VMEM((BSC,), jnp.int32),
            pltpu.VMEM((BSC,), jnp.int32),
            pltpu.VMEM((LANES,), jnp.float32),
            pltpu.VMEM_SHARED((N, DP), jnp.float32),
            pltpu.SemaphoreType.DMA,
            pltpu.SemaphoreType.DMA,
            pltpu.SemaphoreType.DMA,
            pltpu.SemaphoreType.DMA,
            pltpu.SemaphoreType.DMA,
            pltpu.SemaphoreType.DMA,
            pltpu.SemaphoreType.DMA,
            pltpu.SemaphoreType.DMA,
        ],
    )
    def sc(table, srcs3, dsts3, beta16, out,
           ips0, ipd0, ips1, ipd1, srows0, drows0, srows1, drows1,
           obuf0, obuf1, sdidx0, sdidx1, bvec, acc,
           gs0, gd0, gs1, gd1, o0, o1, i0, i1):
        c = lax.axis_index("c")
        s = lax.axis_index("s")
        wid = c * NS + s
        zero16 = jnp.zeros((LANES,), jnp.float32)
        rows = ((srows0, drows0, gs0, gd0), (srows1, drows1, gs1, gd1))
        obufs = ((obuf0, o0, sdidx0), (obuf1, o1, sdidx1))
        ipairs = ((ips0, ipd0, i0), (ips1, ipd1, i1))

        @pl.loop(0, BSC)
        def _zero_obuf(r):
            for k in range(DP // LANES):
                obuf0[r, pl.ds(k * LANES, LANES)] = zero16
                obuf1[r, pl.ds(k * LANES, LANES)] = zero16

        lane = lax.iota(jnp.int32, LANES)

        def copy_sdidx(ipd, half, sd):
            for t in range(2):
                sd[pl.ds(16 * t, 16)] = ipd[pl.ds(half * B + 16 * t, 16)]
            gidx = jnp.minimum(lane + (half * B + 32), half * B + B - 1)
            sd[pl.ds(32, 16)] = plsc.load_gather(ipd, [gidx])

        row0 = s * RPT
        for j in range(NZ):
            pltpu.sync_copy(obuf0, acc.at[pl.ds(row0 + j * BSC, BSC)])
        if REM:
            pltpu.sync_copy(obuf0.at[pl.ds(0, REM)],
                            acc.at[pl.ds(row0 + NZ * BSC, REM)])
        pltpu.sync_copy(beta16, bvec)

        pltpu.sync_copy(srcs3.at[wid, 0], ips0)
        pltpu.sync_copy(dsts3.at[wid, 0], ipd0)
        pltpu.async_copy(srcs3.at[wid, 1], ips1, i1)
        pltpu.async_copy(dsts3.at[wid, 1], ipd1, i1)
        copy_sdidx(ipd0, 0, sdidx0)
        copy_sdidx(ipd0, 0, sdidx1)
        pltpu.async_copy(obuf0, acc.at[sdidx0], o0, add=True)
        pltpu.async_copy(obuf1, acc.at[sdidx1], o1, add=True)
        pltpu.async_copy(table.at[ips0.at[pl.ds(0, B)]], srows0, gs0)
        pltpu.async_copy(table.at[ipd0.at[pl.ds(0, B)]], drows0, gd0)
        plsc.subcore_barrier()

        bs = jnp.max(bvec[...])
        oh0 = (lane == 0).astype(jnp.float32)

        def g_issue(ip, half, p):
            ips, ipd, _ = ip
            sr, dr, ss, sd = rows[p]
            pltpu.async_copy(table.at[ips.at[pl.ds(half * B, B)]], sr, ss)
            pltpu.async_copy(table.at[ipd.at[pl.ds(half * B, B)]], dr, sd)

        def g_wait(ip, half, p):
            ips, ipd, _ = ip
            sr, dr, ss, sd = rows[p]
            pltpu.make_async_copy(table.at[ips.at[pl.ds(half * B, B)]],
                                  sr, ss).wait()
            pltpu.make_async_copy(table.at[ipd.at[pl.ds(half * B, B)]],
                                  dr, sd).wait()

        def i_issue(ip, pairno):
            ips, ipd, isem = ip
            pltpu.async_copy(srcs3.at[wid, pairno], ips, isem)
            pltpu.async_copy(dsts3.at[wid, pairno], ipd, isem)

        def i_wait(ip, pairno):
            ips, ipd, isem = ip
            pltpu.make_async_copy(srcs3.at[wid, pairno], ips, isem).wait()
            pltpu.make_async_copy(dsts3.at[wid, pairno], ipd, isem).wait()

        def compute(ip, half, p):
            _, ipd, _ = ip
            sr, dr, _, _ = rows[p]
            ob, osem, sd = obufs[p]
            pltpu.make_async_copy(ob, acc.at[sd], osem).wait()
            copy_sdidx(ipd, half, sd)

            @pl.loop(0, B, unroll=2)
            def _edge(e):
                def dims(ref):
                    out = []
                    for k in range(PW // LANES):
                        w = plsc.bitcast(ref[e, pl.ds(k * LANES, LANES)],
                                         jnp.bfloat16)
                        lo, hi = plsc.unpack(
                            w, format=plsc.PackFormat.INTERLEAVED,
                            preferred_element_type=jnp.float32)
                        out += [lo, hi]
                    return out

                a = dims(sr)
                b = dims(dr)
                accv = a[0] * b[0]
                for k in range(1, D // LANES):
                    accv = accv + a[k] * b[k]
                dot = jnp.sum(accv)
                wv = jnp.exp(jnp.full((LANES,), bs * dot))
                normv = sr[e, pl.ds(PW, LANES)]
                sv = wv * normv
                for k in range(D // LANES):
                    ob[e, pl.ds(k * LANES, LANES)] = a[k] * sv
                ob[e, pl.ds(D, LANES)] = wv * oh0

            pltpu.async_copy(ob, acc.at[sd], osem, add=True)

        @pl.loop(0, NQUAD)
        def _quad(q):
            ip0, ip1 = ipairs
            g_issue(ip0, 1, 1)
            g_wait(ip0, 0, 0)
            compute(ip0, 0, 0)
            i_wait(ip1, 2 * q + 1)
            g_issue(ip1, 0, 0)
            g_wait(ip0, 1, 1)
            compute(ip0, 1, 1)
            i_issue(ip0, 2 * q + 2)
            g_issue(ip1, 1, 1)
            g_wait(ip1, 0, 0)
            compute(ip1, 0, 0)
            i_wait(ip0, 2 * q + 2)
            g_issue(ip0, 0, 0)
            g_wait(ip1, 1, 1)
            compute(ip1, 1, 1)

            @pl.when(2 * q + 3 < NPAIR)
            def _():
                i_issue(ip1, 2 * q + 3)

        ip0 = ipairs[0]
        g_issue(ip0, 1, 1)
        g_wait(ip0, 0, 0)
        compute(ip0, 0, 0)
        g_wait(ip0, 1, 1)
        compute(ip0, 1, 1)
        pltpu.make_async_copy(obuf0, acc.at[sdidx0], o0).wait()
        pltpu.make_async_copy(obuf1, acc.at[sdidx1], o1).wait()
        plsc.subcore_barrier()
        pltpu.sync_copy(acc.at[pl.ds(row0, RPT)],
                        out.at[pl.ds(c * N + row0, RPT)])

    return sc


def kernel(x, edge_index, beta):
    N = x.shape[0]
    E = edge_index.shape[1]
    NW = 32
    table = pl.pallas_call(
        _norm_body,
        grid=(N // BN,),
        in_specs=[pl.BlockSpec((BN, D), lambda i: (i, 0))],
        out_specs=pl.BlockSpec((BN, TW), lambda i: (i, 0)),
        out_shape=jax.ShapeDtypeStruct((N, TW), jnp.float32),
    )(x)
    npair = (E // NW) // PAIRB
    srcs3 = edge_index[0].reshape(NW, npair, PAIRB)
    dsts3 = edge_index[1].reshape(NW, npair, PAIRB)
    beta16 = jnp.broadcast_to(beta.astype(jnp.float32), (LANES,))
    accflat = _make_sc(N, E)(table, srcs3, dsts3, beta16)
    nb = N // BN
    out = pl.pallas_call(
        _combine_body,
        grid=(nb,),
        in_specs=[pl.BlockSpec((BN, DP), lambda i: (i, 0)),
                  pl.BlockSpec((BN, DP), lambda i: (i + nb, 0))],
        out_specs=pl.BlockSpec((BN, D), lambda i: (i, 0)),
        out_shape=jax.ShapeDtypeStruct((N, D), jnp.float32),
    )(accflat, accflat)
    return out


# speedup vs baseline: 12.1707x; 1.0030x over previous
import functools

import jax
import jax.numpy as jnp
from jax import lax
from jax.experimental import pallas as pl
from jax.experimental.pallas import tpu as pltpu
from jax.experimental.pallas import tpu_sc as plsc

D = 128
TAIL = 16
DP = D + TAIL
PW = D // 2
TW = PW + TAIL
LANES = 16
BN = 1000
B = 40
PAIRB = 2 * B
BSC = 48


def _norm_body(x_ref, out_ref):
    x = x_ref[...]
    nrm = jnp.sqrt(jnp.sum(x * x, axis=1, keepdims=True))
    xn = x / (nrm + 1e-12)
    u = jax.lax.bitcast_convert_type(xn.astype(jnp.bfloat16),
                                     jnp.uint16).astype(jnp.uint32)
    groups = []
    for k in range(4):
        lo = u[:, 32 * k:32 * k + 16]
        hi = u[:, 32 * k + 16:32 * k + 32]
        groups.append((hi << 16) | lo)
    packed = jax.lax.bitcast_convert_type(
        jnp.concatenate(groups, axis=1), jnp.float32)
    tail = jnp.broadcast_to(nrm, (x.shape[0], TAIL))
    out_ref[...] = jnp.concatenate([packed, tail], axis=1)


def _combine_body(a0_ref, a1_ref, out_ref):
    s = a0_ref[...] + a1_ref[...]
    out_ref[...] = s[:, :D] / (s[:, D:D + 1] + 1e-16)


@functools.lru_cache(maxsize=None)
def _make_sc(N, E):
    info = plsc.get_sparse_core_info()
    NC, NS = info.num_cores, info.num_subcores
    NW = NC * NS
    EPW = E // NW
    NCHUNK = EPW // B
    NPAIR = NCHUNK // 2
    NQUAD = (NCHUNK - 2) // 4
    RPT = N // NS
    NZ = RPT // BSC
    REM = RPT - NZ * BSC
    mesh = plsc.VectorSubcoreMesh(core_axis_name="c", subcore_axis_name="s")

    @functools.partial(
        pl.kernel,
        out_type=jax.ShapeDtypeStruct((NC * N, DP), jnp.float32),
        mesh=mesh,
        compiler_params=pltpu.CompilerParams(use_tc_tiling_on_sc=False,
                                             needs_layout_passes=False),
        scratch_types=[
            pltpu.VMEM((PAIRB,), jnp.int32),
            pltpu.VMEM((PAIRB,), jnp.int32),
            pltpu.VMEM((PAIRB,), jnp.int32),
            pltpu.VMEM((PAIRB,), jnp.int32),
            pltpu.VMEM((B, TW), jnp.float32),
            pltpu.VMEM((B, TW), jnp.float32),
            pltpu.VMEM((B, TW), jnp.float32),
            pltpu.VMEM((B, TW), jnp.float32),
            pltpu.VMEM((BSC, DP), jnp.float32),
            pltpu.VMEM((BSC, DP), jnp.float32),
            pltpu.VMEM((BSC,), jnp.int32),
            pltpu.VMEM((BSC,), jnp.int32),
            pltpu.VMEM((LANES,), jnp.float32),
            pltpu.VMEM_SHARED((N, DP), jnp.float32),
            pltpu.SemaphoreType.DMA,
            pltpu.SemaphoreType.DMA,
            pltpu.SemaphoreType.DMA,
            pltpu.SemaphoreType.DMA,
            pltpu.SemaphoreType.DMA,
            pltpu.SemaphoreType.DMA,
            pltpu.SemaphoreType.DMA,
            pltpu.SemaphoreType.DMA,
        ],
    )
    def sc(table, srcs3, dsts3, beta16, out,
           ips0, ipd0, ips1, ipd1, srows0, drows0, srows1, drows1,
           obuf0, obuf1, sdidx0, sdidx1, bvec, acc,
           gs0, gd0, gs1, gd1, o0, o1, i0, i1):
        c = lax.axis_index("c")
        s = lax.axis_index("s")
        wid = c * NS + s
        zero16 = jnp.zeros((LANES,), jnp.float32)
        rows = ((srows0, drows0, gs0, gd0), (srows1, drows1, gs1, gd1))
        obufs = ((obuf0, o0, sdidx0), (obuf1, o1, sdidx1))
        ipairs = ((ips0, ipd0, i0), (ips1, ipd1, i1))

        @pl.loop(0, BSC)
        def _zero_obuf(r):
            for k in range(DP // LANES):
                obuf0[r, pl.ds(k * LANES, LANES)] = zero16
                obuf1[r, pl.ds(k * LANES, LANES)] = zero16

        lane = lax.iota(jnp.int32, LANES)

        def copy_sdidx(ipd, half, sd):
            for t in range(2):
                sd[pl.ds(16 * t, 16)] = ipd[pl.ds(half * B + 16 * t, 16)]
            gidx = jnp.minimum(lane + (half * B + 32), half * B + B - 1)
            sd[pl.ds(32, 16)] = plsc.load_gather(ipd, [gidx])

        row0 = s * RPT
        for j in range(NZ):
            pltpu.sync_copy(obuf0, acc.at[pl.ds(row0 + j * BSC, BSC)])
        if REM:
            pltpu.sync_copy(obuf0.at[pl.ds(0, REM)],
                            acc.at[pl.ds(row0 + NZ * BSC, REM)])
        pltpu.sync_copy(beta16, bvec)

        pltpu.sync_copy(srcs3.at[wid, 0], ips0)
        pltpu.sync_copy(dsts3.at[wid, 0], ipd0)
        pltpu.async_copy(srcs3.at[wid, 1], ips1, i1)
        pltpu.async_copy(dsts3.at[wid, 1], ipd1, i1)
        copy_sdidx(ipd0, 0, sdidx0)
        copy_sdidx(ipd0, 0, sdidx1)
        pltpu.async_copy(obuf0, acc.at[sdidx0], o0, add=True)
        pltpu.async_copy(obuf1, acc.at[sdidx1], o1, add=True)
        pltpu.async_copy(table.at[ips0.at[pl.ds(0, B)]], srows0, gs0)
        pltpu.async_copy(table.at[ipd0.at[pl.ds(0, B)]], drows0, gd0)
        plsc.subcore_barrier()

        bs = jnp.max(bvec[...])
        oh0 = (lane == 0).astype(jnp.float32)

        def g_issue(ip, half, p):
            ips, ipd, _ = ip
            sr, dr, ss, sd = rows[p]
            pltpu.async_copy(table.at[ips.at[pl.ds(half * B, B)]], sr, ss)
            pltpu.async_copy(table.at[ipd.at[pl.ds(half * B, B)]], dr, sd)

        def g_wait(ip, half, p):
            ips, ipd, _ = ip
            sr, dr, ss, sd = rows[p]
            pltpu.make_async_copy(table.at[ips.at[pl.ds(half * B, B)]],
                                  sr, ss).wait()
            pltpu.make_async_copy(table.at[ipd.at[pl.ds(half * B, B)]],
                                  dr, sd).wait()

        def i_issue(ip, pairno):
            ips, ipd, isem = ip
            pltpu.async_copy(srcs3.at[wid, pairno], ips, isem)
            pltpu.async_copy(dsts3.at[wid, pairno], ipd, isem)

        def i_wait(ip, pairno):
            ips, ipd, isem = ip
            pltpu.make_async_copy(srcs3.at[wid, pairno], ips, isem).wait()
            pltpu.make_async_copy(dsts3.at[wid, pairno], ipd, isem).wait()

        def compute(ip, half, p):
            _, ipd, _ = ip
            sr, dr, _, _ = rows[p]
            ob, osem, sd = obufs[p]
            pltpu.make_async_copy(ob, acc.at[sd], osem).wait()
            copy_sdidx(ipd, half, sd)

            @pl.loop(0, B, unroll=4)
            def _edge(e):
                def dims(ref):
                    out = []
                    for k in range(PW // LANES):
                        w = plsc.bitcast(ref[e, pl.ds(k * LANES, LANES)],
                                         jnp.bfloat16)
                        lo, hi = plsc.unpack(
                            w, format=plsc.PackFormat.INTERLEAVED,
                            preferred_element_type=jnp.float32)
                        out += [lo, hi]
                    return out

                a = dims(sr)
                b = dims(dr)
                accv = a[0] * b[0]
                for k in range(1, D // LANES):
                    accv = accv + a[k] * b[k]
                dot = jnp.sum(accv)
                wv = jnp.exp(jnp.full((LANES,), bs * dot))
                normv = sr[e, pl.ds(PW, LANES)]
                sv = wv * normv
                for k in range(D // LANES):
                    ob[e, pl.ds(k * LANES, LANES)] = a[k] * sv
                ob[e, pl.ds(D, LANES)] = wv * oh0

            pltpu.async_copy(ob, acc.at[sd], osem, add=True)

        @pl.loop(0, NQUAD)
        def _quad(q):
            ip0, ip1 = ipairs
            g_issue(ip0, 1, 1)
            g_wait(ip0, 0, 0)
            compute(ip0, 0, 0)
            i_wait(ip1, 2 * q + 1)
            g_issue(ip1, 0, 0)
            g_wait(ip0, 1, 1)
            compute(ip0, 1, 1)
            i_issue(ip0, 2 * q + 2)
            g_issue(ip1, 1, 1)
            g_wait(ip1, 0, 0)
            compute(ip1, 0, 0)
            i_wait(ip0, 2 * q + 2)
            g_issue(ip0, 0, 0)
            g_wait(ip1, 1, 1)
            compute(ip1, 1, 1)

            @pl.when(2 * q + 3 < NPAIR)
            def _():
                i_issue(ip1, 2 * q + 3)

        ip0 = ipairs[0]
        g_issue(ip0, 1, 1)
        g_wait(ip0, 0, 0)
        compute(ip0, 0, 0)
        g_wait(ip0, 1, 1)
        compute(ip0, 1, 1)
        pltpu.make_async_copy(obuf0, acc.at[sdidx0], o0).wait()
        pltpu.make_async_copy(obuf1, acc.at[sdidx1], o1).wait()
        plsc.subcore_barrier()
        pltpu.sync_copy(acc.at[pl.ds(row0, RPT)],
                        out.at[pl.ds(c * N + row0, RPT)])

    return sc


def kernel(x, edge_index, beta):
    N = x.shape[0]
    E = edge_index.shape[1]
    NW = 32
    table = pl.pallas_call(
        _norm_body,
        grid=(N // BN,),
        in_specs=[pl.BlockSpec((BN, D), lambda i: (i, 0))],
        out_specs=pl.BlockSpec((BN, TW), lambda i: (i, 0)),
        out_shape=jax.ShapeDtypeStruct((N, TW), jnp.float32),
    )(x)
    npair = (E // NW) // PAIRB
    srcs3 = edge_index[0].reshape(NW, npair, PAIRB)
    dsts3 = edge_index[1].reshape(NW, npair, PAIRB)
    beta16 = jnp.broadcast_to(beta.astype(jnp.float32), (LANES,))
    accflat = _make_sc(N, E)(table, srcs3, dsts3, beta16)
    nb = N // BN
    out = pl.pallas_call(
        _combine_body,
        grid=(nb,),
        in_specs=[pl.BlockSpec((BN, DP), lambda i: (i, 0)),
                  pl.BlockSpec((BN, DP), lambda i: (i + nb, 0))],
        out_specs=pl.BlockSpec((BN, D), lambda i: (i, 0)),
        out_shape=jax.ShapeDtypeStruct((N, D), jnp.float32),
    )(accflat, accflat)
    return out
